# baseline (speedup 1.0000x reference)
import functools

import jax
import jax.numpy as jnp
from jax.experimental import pallas as pl
from jax.experimental.pallas import tpu as pltpu

_NK = 8


def _fused_kernel(x_ref, w_ref, b_ref, o_ref, acc_ref, *, inv_s, nk):
    k = pl.program_id(1)
    part = jnp.sum(x_ref[...], axis=1, dtype=jnp.float32)

    @pl.when(k == 0)
    def _():
        acc_ref[...] = part

    @pl.when(k > 0)
    def _():
        acc_ref[...] += part

    @pl.when(k == nk - 1)
    def _():
        mean = acc_ref[...] * inv_s
        y = jnp.dot(mean, w_ref[...], preferred_element_type=jnp.float32)
        o_ref[...] = (y + b_ref[...]).astype(o_ref.dtype)


def kernel(x, weight, bias):
    B, S, C_in = x.shape
    C_out = weight.shape[-1]
    out_dtype = x.dtype
    inv_s = 1.0 / float(S)
    itemsize = x.dtype.itemsize

    tb = B // 2
    nk = _NK
    ts = S // nk

    x_block_bytes = tb * ts * C_in * itemsize
    vmem_limit = int(min(2 * x_block_bytes + (16 << 20), 100 << 20))

    cost = pl.CostEstimate(
        flops=B * S * C_in + 2 * B * C_in * C_out,
        transcendentals=0,
        bytes_accessed=x.size * itemsize + weight.size * 4 + B * C_out * 4,
    )

    w = weight.astype(jnp.float32)
    b2d = bias.astype(jnp.float32).reshape(1, C_out)

    return pl.pallas_call(
        functools.partial(_fused_kernel, inv_s=inv_s, nk=nk),
        out_shape=jax.ShapeDtypeStruct((B, C_out), out_dtype),
        grid=(2, nk),
        in_specs=[
            pl.BlockSpec((tb, ts, C_in), lambda c, k: (c, k, 0)),
            pl.BlockSpec((C_in, C_out), lambda c, k: (0, 0)),
            pl.BlockSpec((1, C_out), lambda c, k: (0, 0)),
        ],
        out_specs=pl.BlockSpec((tb, C_out), lambda c, k: (c, 0)),
        scratch_shapes=[pltpu.VMEM((tb, C_in), jnp.float32)],
        compiler_params=pltpu.CompilerParams(
            dimension_semantics=("parallel", "arbitrary"),
            vmem_limit_bytes=vmem_limit,
        ),
        cost_estimate=cost,
    )(x, w, b2d)


# speedup vs baseline: 1.0575x; 1.0575x over previous
import functools

import jax
import jax.numpy as jnp
from jax.experimental import pallas as pl
from jax.experimental.pallas import tpu as pltpu

_TB = 24


def _fused_kernel(x_ref, w_ref, b_ref, o_ref):
    s = jnp.sum(x_ref[...], axis=1, dtype=jnp.float32)
    y = jnp.dot(s, w_ref[...], preferred_element_type=jnp.float32)
    o_ref[...] = (y + b_ref[...]).astype(o_ref.dtype)


def kernel(x, weight, bias):
    B, S, C_in = x.shape
    C_out = weight.shape[-1]
    out_dtype = x.dtype
    itemsize = x.dtype.itemsize

    tb = _TB
    nb = -(-B // tb)

    x_block_bytes = tb * S * C_in * itemsize
    vmem_limit = int(min(2 * x_block_bytes + (8 << 20), 100 << 20))

    cost = pl.CostEstimate(
        flops=B * S * C_in + 2 * B * C_in * C_out,
        transcendentals=0,
        bytes_accessed=x.size * itemsize + weight.size * 4 + B * C_out * 4,
    )

    w = weight.astype(jnp.float32) * (1.0 / float(S))
    b2d = bias.astype(jnp.float32).reshape(1, C_out)

    return pl.pallas_call(
        _fused_kernel,
        out_shape=jax.ShapeDtypeStruct((B, C_out), out_dtype),
        grid=(nb,),
        in_specs=[
            pl.BlockSpec((tb, S, C_in), lambda i: (i, 0, 0)),
            pl.BlockSpec((C_in, C_out), lambda i: (0, 0)),
            pl.BlockSpec((1, C_out), lambda i: (0, 0)),
        ],
        out_specs=pl.BlockSpec((tb, C_out), lambda i: (i, 0)),
        compiler_params=pltpu.CompilerParams(
            dimension_semantics=("parallel",),
            vmem_limit_bytes=vmem_limit,
        ),
        cost_estimate=cost,
    )(x, w, b2d)


# speedup vs baseline: 1.0597x; 1.0021x over previous
import functools

import jax
import jax.numpy as jnp
from jax.experimental import pallas as pl
from jax.experimental.pallas import tpu as pltpu

_TB = 24


def _fused_kernel(x_ref, w_ref, b_ref, o_ref):
    s = jnp.sum(x_ref[...], axis=1, dtype=jnp.float32)
    y = jnp.dot(s, w_ref[...], preferred_element_type=jnp.float32)
    o_ref[...] = (y + b_ref[...]).astype(o_ref.dtype)


def kernel(x, weight, bias):
    B, S, C_in = x.shape
    C_out = weight.shape[-1]
    out_dtype = x.dtype
    itemsize = x.dtype.itemsize

    tb = _TB
    nb = -(-B // tb)

    vmem_limit = 32 << 20

    cost = pl.CostEstimate(
        flops=B * S * C_in,
        transcendentals=0,
        bytes_accessed=x.size * itemsize + B * C_in * 4,
    )

    w = weight.astype(jnp.float32) * (1.0 / float(S))
    b2d = bias.astype(jnp.float32).reshape(1, C_out)

    return pl.pallas_call(
        _fused_kernel,
        out_shape=jax.ShapeDtypeStruct((B, C_out), out_dtype),
        grid_spec=pltpu.PrefetchScalarGridSpec(
            num_scalar_prefetch=0,
            grid=(nb, 1),
            in_specs=[
                pl.BlockSpec((tb, S, C_in), lambda i, k: (i, 0, 0)),
                pl.BlockSpec((C_in, C_out), lambda i, k: (0, 0)),
                pl.BlockSpec((1, C_out), lambda i, k: (0, 0)),
            ],
            out_specs=pl.BlockSpec((tb, C_out), lambda i, k: (i, 0)),
        ),
        compiler_params=pltpu.CompilerParams(
            dimension_semantics=("parallel", "arbitrary"),
            vmem_limit_bytes=vmem_limit,
        ),
        cost_estimate=cost,
    )(x, w, b2d)


# speedup vs baseline: 1.0603x; 1.0005x over previous
import jax
import jax.numpy as jnp
from jax.experimental import pallas as pl
from jax.experimental.pallas import tpu as pltpu

_TB = 24


def _fused_kernel(x_ref, w_ref, b_ref, o_ref):
    s = jnp.sum(x_ref[...], axis=1, dtype=jnp.float32)
    y = jnp.dot(s, w_ref[...], preferred_element_type=jnp.float32)
    o_ref[...] = (y + b_ref[...]).astype(o_ref.dtype)


def kernel(x, weight, bias):
    B, S, C_in = x.shape
    C_out = weight.shape[-1]
    out_dtype = x.dtype
    itemsize = x.dtype.itemsize

    tb = _TB
    nb = -(-B // tb)

    vmem_limit = 32 << 20

    cost = pl.CostEstimate(
        flops=B * S * C_in,
        transcendentals=0,
        bytes_accessed=x.size * itemsize + B * C_in * 4,
    )

    w = weight.astype(jnp.float32) * (1.0 / float(S))
    b2d = bias.astype(jnp.float32).reshape(1, C_out)

    return pl.pallas_call(
        _fused_kernel,
        out_shape=jax.ShapeDtypeStruct((B, C_out), out_dtype),
        grid_spec=pltpu.PrefetchScalarGridSpec(
            num_scalar_prefetch=0,
            grid=(nb, 1),
            in_specs=[
                pl.BlockSpec((tb, S, C_in), lambda i, k: (i, 0, 0)),
                pl.BlockSpec((C_in, C_out), lambda i, k: (0, 0)),
                pl.BlockSpec((1, C_out), lambda i, k: (0, 0)),
            ],
            out_specs=pl.BlockSpec((tb, C_out), lambda i, k: (i, 0)),
        ),
        compiler_params=pltpu.CompilerParams(
            dimension_semantics=("parallel", "arbitrary"),
            vmem_limit_bytes=vmem_limit,
        ),
        cost_estimate=cost,
    )(x, w, b2d)
